# Initial kernel scaffold; baseline (speedup 1.0000x reference)
import jax
import jax.numpy as jnp
from jax.experimental import pallas as pl


def kernel(x, edge_index, batch, conv1_Wl, conv1_bl, conv1_Wr, conv2_Wl, conv2_bl, conv2_Wr, conv4_W, conv4_b, conv5_W, conv5_b, p1, p2, p4, p5, lin1_W, lin1_b, lin2_W, lin2_b, lin3_W, lin3_b):
    raise NotImplementedError("write your pallas kernel here")


# speedup vs baseline: 1.0235x; 1.0235x over previous
import functools
import math

import jax
import jax.numpy as jnp
from jax.experimental import pallas as pl
from jax.experimental.pallas import tpu as pltpu

_CW = 64
_RATIO = 0.5


def _seg_sums(table, src, dst, mask, n):
    gsrc = jnp.where(mask > 0, src, n)
    rows = table[gsrc]
    out = jax.ops.segment_sum(rows, dst, num_segments=n + 1)
    return out


def _sage(h, src, dst, mask, Wl, bl, Wr, n, d_in):
    W = 16 if d_in <= 2 else 80
    table = jnp.zeros((n + 1, W), jnp.float32)
    table = table.at[:n, :d_in].set(h)
    table = table.at[:n, d_in].set(1.0)
    out = _seg_sums(table, src, dst, mask, n)
    agg = out[:n, :d_in]
    deg = out[:n, d_in]
    mean = agg / jnp.maximum(deg, 1.0)[:, None]
    return mean @ Wl.T + bl + h @ Wr.T


def _gcn(h, src, dst, mask, Wc, b, n):
    xw = h @ Wc.T
    W = 80
    table = jnp.zeros((n + 1, W), jnp.float32)
    table = table.at[:n, :_CW].set(xw)
    table = table.at[:n, W - 16].set(1.0)
    deg = jax.ops.segment_sum(mask, dst, num_segments=n) + 1.0
    dinv = 1.0 / jnp.sqrt(deg)
    norm = dinv[src] * dinv[dst] * mask
    out = jax.ops.segment_sum(xw[src] * norm[:, None], dst, num_segments=n)
    out = out + xw * (dinv * dinv)[:, None]
    return out + b


def _pool(h, src, dst, mask, p, n):
    score = jnp.tanh((h @ p) / jnp.linalg.norm(p))
    k = int(math.ceil(_RATIO * n))
    vals, perm = jax.lax.top_k(score, k)
    h_new = h[perm] * vals[:, None]
    new_idx = jnp.full((n,), -1, jnp.int32).at[perm].set(
        jnp.arange(k, dtype=jnp.int32))
    ns = new_idx[src]
    nd = new_idx[dst]
    new_mask = mask * (ns >= 0).astype(h.dtype) * (nd >= 0).astype(h.dtype)
    ns = jnp.where(ns >= 0, ns, 0)
    nd = jnp.where(nd >= 0, nd, 0)
    return h_new, ns, nd, new_mask, k


def _head_kernel(h1, h2, h4, h5, w1, b1, w2, b2, w3, b3, o_ref):
    def ro(ref, k):
        rows = ref.shape[0]
        idx = jax.lax.broadcasted_iota(jnp.int32, (rows, 1), 0)
        valid = idx < k
        x = ref[...]
        mx = jnp.max(jnp.where(valid, x, -jnp.inf), axis=0, keepdims=True)
        mn = jnp.sum(jnp.where(valid, x, 0.0), axis=0, keepdims=True) / k
        return jnp.concatenate([mx, mn], axis=1)

    z = (ro(h1, 25000) + ro(h2, 12500) + ro(h4, 6250) + ro(h5, 3125))
    z = jax.nn.relu(z @ w1[...].T + b1[...][None, :])
    z = jax.nn.relu(z @ w2[...].T + b2[...][None, :])
    z = z @ w3[...].T + b3[...][None, :]
    z = z - jnp.max(z, axis=1, keepdims=True)
    e = jnp.exp(z)
    o_ref[...] = e / jnp.sum(e, axis=1, keepdims=True)


def _head(h1, h2, h4, h5, w1, b1, w2, b2, w3, b3):
    def pad(h):
        r = (-h.shape[0]) % 8
        return jnp.pad(h, ((0, r), (0, 0)))

    return pl.pallas_call(
        _head_kernel,
        out_shape=jax.ShapeDtypeStruct((1, 2), jnp.float32),
    )(pad(h1), pad(h2), pad(h4), pad(h5), w1, b1, w2, b2, w3, b3)


def kernel(x, edge_index, batch, conv1_Wl, conv1_bl, conv1_Wr, conv2_Wl,
           conv2_bl, conv2_Wr, conv4_W, conv4_b, conv5_W, conv5_b, p1, p2,
           p4, p5, lin1_W, lin1_b, lin2_W, lin2_b, lin3_W, lin3_b):
    src = edge_index[0]
    dst = edge_index[1]
    mask = jnp.ones((src.shape[0],), jnp.float32)
    n = x.shape[0]
    h = jax.nn.relu(_sage(x, src, dst, mask, conv1_Wl, conv1_bl, conv1_Wr, n, 2))
    h, src, dst, mask, n = _pool(h, src, dst, mask, p1, n)
    h1 = h
    h = jax.nn.relu(_sage(h, src, dst, mask, conv2_Wl, conv2_bl, conv2_Wr, n, _CW))
    h, src, dst, mask, n = _pool(h, src, dst, mask, p2, n)
    h2 = h
    h = jax.nn.relu(_gcn(h, src, dst, mask, conv4_W, conv4_b, n))
    h, src, dst, mask, n = _pool(h, src, dst, mask, p4, n)
    h4 = h
    h = jax.nn.relu(_gcn(h, src, dst, mask, conv5_W, conv5_b, n))
    h, src, dst, mask, n = _pool(h, src, dst, mask, p5, n)
    h5 = h
    return _head(h1, h2, h4, h5, lin1_W, lin1_b, lin2_W, lin2_b,
                 lin3_W, lin3_b)


# speedup vs baseline: 10.6540x; 10.4094x over previous
import functools
import math

import jax
import jax.numpy as jnp
from jax import lax
from jax.experimental import pallas as pl
from jax.experimental.pallas import tpu as pltpu
from jax.experimental.pallas import tpu_sc as plsc

_N = 50000
_E = 800000
_CW = 64

_NC = 2
_NS = 16
_HALF = _N // 2
_Z = 64
_RPT = ((_HALF + 1 + _NS * _Z - 1) // (_NS * _Z)) * _Z
_R = _RPT * _NS
_EPT = _E // _NS
_C = 128
_NFULL = _EPT // _C
_TAIL = _EPT - _NFULL * _C


def _edge_pass_body(src_hbm, dst_hbm, table_hbm, out_hbm,
                    spbuf, csrc, cdst, ldst, rows, zbuf, sem, *, width):
    c = lax.axis_index("c")
    s = lax.axis_index("s")
    half = _HALF

    for r in range(_Z):
        for w in range(width // 16):
            zbuf[r, pl.ds(w * 16, 16)] = jnp.zeros((16,), jnp.float32)
    for z in range(_RPT // _Z):
        pltpu.sync_copy(zbuf, spbuf.at[pl.ds(s * _RPT + z * _Z, _Z)])
    plsc.subcore_barrier()

    ebase = s * _EPT

    def do_chunk(off, ngroups):
        pltpu.sync_copy(src_hbm.at[pl.ds(off, ngroups * 16)],
                        csrc.at[pl.ds(0, ngroups * 16)])
        pltpu.sync_copy(dst_hbm.at[pl.ds(off, ngroups * 16)],
                        cdst.at[pl.ds(0, ngroups * 16)])
        for g in range(8):
            if g < ngroups:
                d16 = cdst[pl.ds(g * 16, 16)]
                ld = d16 - c * half
                inr = (ld >= 0) & (ld < half)
                ldst[pl.ds(g * 16, 16)] = jnp.where(
                    inr, ld, jnp.full((16,), half, jnp.int32))
            else:
                csrc[pl.ds(g * 16, 16)] = jnp.zeros((16,), jnp.int32)
                ldst[pl.ds(g * 16, 16)] = jnp.full((16,), half, jnp.int32)
        pltpu.async_copy(table_hbm.at[csrc], rows, sem).wait()
        pltpu.sync_copy(rows, spbuf.at[ldst], add=True)

    def chunk_body(i, carry):
        do_chunk(ebase + i * _C, 8)
        return carry

    lax.fori_loop(0, _NFULL, chunk_body, 0)
    if _TAIL:
        do_chunk(ebase + _NFULL * _C, _TAIL // 16)

    plsc.subcore_barrier()
    for z in range(_RPT // _Z):
        r0 = s * _RPT + z * _Z
        pltpu.sync_copy(spbuf.at[pl.ds(r0, _Z)],
                        out_hbm.at[pl.ds(c * _R + r0, _Z)])


@functools.partial(jax.jit, static_argnames=("width",))
def _edge_pass(src, dst, table, width):
    mesh = plsc.VectorSubcoreMesh(core_axis_name="c", subcore_axis_name="s")
    body = functools.partial(_edge_pass_body, width=width)
    out = pl.kernel(
        body,
        out_type=jax.ShapeDtypeStruct((_NC * _R, width), jnp.float32),
        mesh=mesh,
        scratch_types=[
            pltpu.VMEM_SHARED((_R, width), jnp.float32),
            pltpu.VMEM((_C,), jnp.int32),
            pltpu.VMEM((_C,), jnp.int32),
            pltpu.VMEM((_C,), jnp.int32),
            pltpu.VMEM((_C, width), jnp.float32),
            pltpu.VMEM((_Z, width), jnp.float32),
            pltpu.SemaphoreType.DMA,
        ],
        compiler_params=pltpu.CompilerParams(use_tc_tiling_on_sc=False),
    )(src, dst, table)
    return jnp.concatenate([out[:_HALF], out[_R:_R + _HALF]], axis=0)


def _select(h, p, alive, k):
    score = jnp.tanh((h @ p) / jnp.linalg.norm(p))
    key = jnp.where(alive > 0, score, -jnp.inf)
    _, perm = jax.lax.top_k(key, k)
    new_alive = jnp.zeros((_N,), jnp.float32).at[perm].set(1.0)
    return h * score[:, None] * new_alive[:, None], new_alive


def _readout(h, alive, k):
    mx = jnp.max(jnp.where(alive[:, None] > 0, h, -jnp.inf), axis=0,
                 keepdims=True)
    mn = jnp.sum(h, axis=0, keepdims=True) / k
    return jnp.concatenate([mx, mn], axis=1)


def _head_kernel(z, w1, b1, w2, b2, w3, b3, o_ref):
    v = z[...]
    v = jax.nn.relu(v @ w1[...].T + b1[...][None, :])
    v = jax.nn.relu(v @ w2[...].T + b2[...][None, :])
    v = v @ w3[...].T + b3[...][None, :]
    v = v - jnp.max(v, axis=1, keepdims=True)
    e = jnp.exp(v)
    o_ref[...] = e / jnp.sum(e, axis=1, keepdims=True)


def _head(z, w1, b1, w2, b2, w3, b3):
    return pl.pallas_call(
        _head_kernel,
        out_shape=jax.ShapeDtypeStruct((1, 2), jnp.float32),
    )(z, w1, b1, w2, b2, w3, b3)


def kernel(x, edge_index, batch, conv1_Wl, conv1_bl, conv1_Wr, conv2_Wl,
           conv2_bl, conv2_Wr, conv4_W, conv4_b, conv5_W, conv5_b, p1, p2,
           p4, p5, lin1_W, lin1_b, lin2_W, lin2_b, lin3_W, lin3_b):
    src = edge_index[0]
    dst = edge_index[1]

    t1 = jnp.concatenate(
        [x, jnp.ones((_N, 1), jnp.float32), jnp.zeros((_N, 13), jnp.float32)],
        axis=1)
    o1 = _edge_pass(src, dst, t1, 16)
    agg = o1[:, :2]
    deg = o1[:, 2]
    mean = agg / jnp.maximum(deg, 1.0)[:, None]
    h = jax.nn.relu(mean @ conv1_Wl.T + conv1_bl + x @ conv1_Wr.T)
    h, alive = _select(h, p1, jnp.ones((_N,), jnp.float32), 25000)
    z = _readout(h, alive, 25000)

    def deg_of(alive_now):
        td = jnp.concatenate(
            [alive_now[:, None], jnp.zeros((_N, 15), jnp.float32)], axis=1)
        return _edge_pass(src, dst, td, 16)[:, 0]

    deg = deg_of(alive)
    agg = _edge_pass(src, dst, h, 64)
    mean = agg / jnp.maximum(deg, 1.0)[:, None]
    h = jax.nn.relu(mean @ conv2_Wl.T + conv2_bl + h @ conv2_Wr.T)
    h, alive = _select(h, p2, alive, 12500)
    z = z + _readout(h, alive, 12500)

    def gcn(h_in, alive_now, Wc, b):
        deg_n = deg_of(alive_now) + 1.0
        dinv = lax.rsqrt(deg_n)
        xw = h_in @ Wc.T
        agg_n = _edge_pass(src, dst, xw * dinv[:, None] * alive_now[:, None],
                           64)
        return agg_n * dinv[:, None] + xw * (dinv * dinv)[:, None] + b

    h = jax.nn.relu(gcn(h, alive, conv4_W, conv4_b))
    h, alive = _select(h, p4, alive, 6250)
    z = z + _readout(h, alive, 6250)

    h = jax.nn.relu(gcn(h, alive, conv5_W, conv5_b))
    h, alive = _select(h, p5, alive, 3125)
    z = z + _readout(h, alive, 3125)

    return _head(z, lin1_W, lin1_b, lin2_W, lin2_b, lin3_W, lin3_b)
